# Initial kernel scaffold; baseline (speedup 1.0000x reference)
import jax
import jax.numpy as jnp
from jax.experimental import pallas as pl


def kernel(x, edge_index, Wl0, bl0, Wr0, Wl1, bl1, Wr1, Wl2, bl2, Wr2, W1, b1, W2, b2):
    raise NotImplementedError("write your pallas kernel here")


# speedup vs baseline: 6.3906x; 6.3906x over previous
import functools

import jax
import jax.numpy as jnp
from jax import lax
from jax.experimental import pallas as pl
from jax.experimental.pallas import tpu as pltpu
from jax.experimental.pallas import tpu_sc as plsc

N = 10000
E = 320000
D = 128
NC = 2
NS = 16
NW = NC * NS
EPW = E // NW
K = 80
NCHUNK = EPW // K
RPT = 624
RREM = N - NS * RPT
CW = 128

_mesh = plsc.VectorSubcoreMesh(core_axis_name="c", subcore_axis_name="s")


def _lane_perm(v, idx):
    return lax.gather(
        v, idx[:, None],
        lax.GatherDimensionNumbers(offset_dims=(), collapsed_slice_dims=(0,),
                                   start_index_map=(0,)),
        slice_sizes=(1,),
        mode=lax.GatherScatterMode.PROMISE_IN_BOUNDS)


def _seg_body(h_hbm, src_hbm, dst_hbm, z_hbm, parts_out,
              src_v, dst_v, rows_v, sem, acc_sh):
    cid = lax.axis_index("c")
    sid = lax.axis_index("s")
    wid = sid * NC + cid
    r0 = sid * RPT

    def _rowcopy(src_mem, dst_mem):
        pltpu.sync_copy(src_mem.at[pl.ds(r0, RPT)], dst_mem.at[pl.ds(r0, RPT)])

        @pl.when(sid == NS - 1)
        def _():
            pltpu.sync_copy(src_mem.at[pl.ds(NS * RPT, RREM)],
                            dst_mem.at[pl.ds(NS * RPT, RREM)])

    _rowcopy(z_hbm, acc_sh)
    pltpu.sync_copy(src_hbm.at[wid], src_v)
    pltpu.sync_copy(dst_hbm.at[wid], dst_v)
    plsc.subcore_barrier()

    @pl.loop(0, NCHUNK)
    def _chunk(i):
        pltpu.async_copy(h_hbm.at[src_v.at[i]], rows_v, sem).wait()
        pltpu.sync_copy(rows_v, acc_sh.at[dst_v.at[i]], add=True)

    plsc.subcore_barrier()
    _rowcopy(acc_sh, parts_out.at[cid])


_seg_sum = functools.partial(
    pl.kernel,
    _seg_body,
    out_type=jax.ShapeDtypeStruct((NC, N, D), jnp.float32),
    mesh=_mesh,
    scratch_types=[
        pltpu.VMEM((NCHUNK, K), jnp.int32),
        pltpu.VMEM((NCHUNK, K), jnp.int32),
        pltpu.VMEM((K, D), jnp.float32),
        pltpu.SemaphoreType.DMA,
        pltpu.VMEM_SHARED((N, D), jnp.float32),
    ],
)()


def _cnt_body(dst_hbm, zc_hbm, ones_hbm, cnt_out,
              dst_v, ones_v, cnt_sh):
    cid = lax.axis_index("c")
    sid = lax.axis_index("s")
    wid = sid * NC + cid
    r0 = sid * RPT

    def _rowcopy(src_mem, dst_mem):
        pltpu.sync_copy(src_mem.at[pl.ds(r0, RPT)], dst_mem.at[pl.ds(r0, RPT)])

        @pl.when(sid == NS - 1)
        def _():
            pltpu.sync_copy(src_mem.at[pl.ds(NS * RPT, RREM)],
                            dst_mem.at[pl.ds(NS * RPT, RREM)])

    _rowcopy(zc_hbm, cnt_sh)
    pltpu.sync_copy(dst_hbm.at[wid], dst_v)
    pltpu.sync_copy(ones_hbm, ones_v)
    plsc.subcore_barrier()

    @pl.loop(0, NCHUNK)
    def _chunk(i):
        pltpu.sync_copy(ones_v, cnt_sh.at[dst_v.at[i]], add=True)

    plsc.subcore_barrier()
    _rowcopy(cnt_sh, cnt_out.at[cid])


_cnt_sum = functools.partial(
    pl.kernel,
    _cnt_body,
    out_type=jax.ShapeDtypeStruct((NC, N, CW), jnp.float32),
    mesh=_mesh,
    scratch_types=[
        pltpu.VMEM((NCHUNK, K), jnp.int32),
        pltpu.VMEM((K, CW), jnp.float32),
        pltpu.VMEM_SHARED((N, CW), jnp.float32),
    ],
)()


BN = 400


def _tc_layer_body(p_ref, c_ref, h_ref, wl_ref, bl_ref, wr_ref, o_ref):
    cnt = c_ref[0, :, 0:1] + c_ref[1, :, 0:1]
    aggr = (p_ref[0] + p_ref[1]) / jnp.maximum(cnt, 1.0)
    o_ref[...] = jnp.maximum(
        jnp.dot(aggr, wl_ref[...], preferred_element_type=jnp.float32)
        + bl_ref[...]
        + jnp.dot(h_ref[...], wr_ref[...], preferred_element_type=jnp.float32),
        0.0)


def _tc_layer(parts, cnt_parts, h, WlT, bl, WrT):
    return pl.pallas_call(
        _tc_layer_body,
        grid=(N // BN,),
        in_specs=[
            pl.BlockSpec((NC, BN, D), lambda i: (0, i, 0)),
            pl.BlockSpec((NC, BN, CW), lambda i: (0, i, 0)),
            pl.BlockSpec((BN, D), lambda i: (i, 0)),
            pl.BlockSpec((D, D), lambda i: (0, 0)),
            pl.BlockSpec((1, D), lambda i: (0, 0)),
            pl.BlockSpec((D, D), lambda i: (0, 0)),
        ],
        out_specs=pl.BlockSpec((BN, D), lambda i: (i, 0)),
        out_shape=jax.ShapeDtypeStruct((N, D), jnp.float32),
    )(parts, cnt_parts, h, WlT, bl, WrT)


def _tc_final_body(p_ref, c_ref, h_ref, wl_ref, bl_ref, wr_ref,
                   w1a_ref, w1b_ref, b1_ref, a_ref, b_ref):
    cnt = c_ref[0, :, 0:1] + c_ref[1, :, 0:1]
    aggr = (p_ref[0] + p_ref[1]) / jnp.maximum(cnt, 1.0)
    h3 = jnp.maximum(
        jnp.dot(aggr, wl_ref[...], preferred_element_type=jnp.float32)
        + bl_ref[...]
        + jnp.dot(h_ref[...], wr_ref[...], preferred_element_type=jnp.float32),
        0.0)
    a_ref[...] = jnp.dot(h3, w1a_ref[...], preferred_element_type=jnp.float32) + b1_ref[...]
    b_ref[...] = jnp.dot(h3, w1b_ref[...], preferred_element_type=jnp.float32)


def _tc_final(parts, cnt_parts, h, WlT, bl, WrT, W1aT, W1bT, b1):
    return pl.pallas_call(
        _tc_final_body,
        grid=(N // BN,),
        in_specs=[
            pl.BlockSpec((NC, BN, D), lambda i: (0, i, 0)),
            pl.BlockSpec((NC, BN, CW), lambda i: (0, i, 0)),
            pl.BlockSpec((BN, D), lambda i: (i, 0)),
            pl.BlockSpec((D, D), lambda i: (0, 0)),
            pl.BlockSpec((1, D), lambda i: (0, 0)),
            pl.BlockSpec((D, D), lambda i: (0, 0)),
            pl.BlockSpec((D, D), lambda i: (0, 0)),
            pl.BlockSpec((D, D), lambda i: (0, 0)),
            pl.BlockSpec((1, D), lambda i: (0, 0)),
        ],
        out_specs=[pl.BlockSpec((BN, D), lambda i: (i, 0)),
                   pl.BlockSpec((BN, D), lambda i: (i, 0))],
        out_shape=[jax.ShapeDtypeStruct((N, D), jnp.float32),
                   jax.ShapeDtypeStruct((N, D), jnp.float32)],
    )(parts, cnt_parts, h, WlT, bl, WrT, W1aT, W1bT, b1)


def _edge_body(a_hbm, b_hbm, src_hbm, dst_hbm, w2_hbm, b2_hbm, out_hbm,
               src_v, dst_v, a_rows, b_rows, w2_v, b2_v, out_v, sem_a, sem_b):
    cid = lax.axis_index("c")
    sid = lax.axis_index("s")
    wid = sid * NC + cid
    pltpu.sync_copy(src_hbm.at[wid], src_v)
    pltpu.sync_copy(dst_hbm.at[wid], dst_v)
    pltpu.sync_copy(w2_hbm, w2_v)
    pltpu.sync_copy(b2_hbm, b2_v)
    w2r = [w2_v[pl.ds(k * 16, 16)] for k in range(D // 16)]
    lane = lax.iota(jnp.int32, 16)

    @pl.loop(0, NCHUNK)
    def _chunk(c):
        ca = pltpu.async_copy(a_hbm.at[src_v.at[c]], a_rows, sem_a)
        cb = pltpu.async_copy(b_hbm.at[dst_v.at[c]], b_rows, sem_b)
        ca.wait()
        cb.wait()
        for g in range(K // 16):
            def _edge(j, acc):
                e = g * 16 + j
                t = jnp.zeros((16,), jnp.float32)
                for k in range(D // 16):
                    av = a_rows[e, pl.ds(k * 16, 16)]
                    bv = b_rows[e, pl.ds(k * 16, 16)]
                    t = t + jnp.maximum(av + bv, 0.0) * w2r[k]
                for sh in (8, 4, 2, 1):
                    t = t + _lane_perm(t, jnp.bitwise_xor(lane, sh))
                return jnp.where(lane == j, t, acc)

            acc = lax.fori_loop(0, 16, _edge, jnp.zeros((16,), jnp.float32))
            out_v[pl.ds(c * K + g * 16, 16)] = acc + b2_v[...]

    pltpu.sync_copy(out_v, out_hbm.at[pl.ds(wid * EPW, EPW)])


_edge_mlp = functools.partial(
    pl.kernel,
    _edge_body,
    out_type=jax.ShapeDtypeStruct((E,), jnp.float32),
    mesh=_mesh,
    scratch_types=[
        pltpu.VMEM((NCHUNK, K), jnp.int32),
        pltpu.VMEM((NCHUNK, K), jnp.int32),
        pltpu.VMEM((K, D), jnp.float32),
        pltpu.VMEM((K, D), jnp.float32),
        pltpu.VMEM((D,), jnp.float32),
        pltpu.VMEM((16,), jnp.float32),
        pltpu.VMEM((EPW,), jnp.float32),
        pltpu.SemaphoreType.DMA,
        pltpu.SemaphoreType.DMA,
    ],
)()


def kernel(x, edge_index, Wl0, bl0, Wr0, Wl1, bl1, Wr1, Wl2, bl2, Wr2, W1, b1, W2, b2):
    src = edge_index[0].reshape(NW, NCHUNK, K)
    dst = edge_index[1].reshape(NW, NCHUNK, K)
    z = jnp.zeros((N, D), jnp.float32)
    zc = jnp.zeros((N, CW), jnp.float32)
    ones = jnp.ones((K, CW), jnp.float32)

    cnt_parts = _cnt_sum(dst, zc, ones)
    parts0 = _seg_sum(x, src, dst, z)
    h1 = _tc_layer(parts0, cnt_parts, x, Wl0.T, bl0.reshape(1, D), Wr0.T)
    parts1 = _seg_sum(h1, src, dst, z)
    h2 = _tc_layer(parts1, cnt_parts, h1, Wl1.T, bl1.reshape(1, D), Wr1.T)
    parts2 = _seg_sum(h2, src, dst, z)
    W1T = W1.T
    A, B = _tc_final(parts2, cnt_parts, h2, Wl2.T, bl2.reshape(1, D), Wr2.T,
                     W1T[:D], W1T[D:], b1.reshape(1, D))
    out = _edge_mlp(A, B, src, dst, W2.reshape(D), jnp.broadcast_to(b2, (16,)))
    return out


# speedup vs baseline: 7.0068x; 1.0964x over previous
import functools

import jax
import jax.numpy as jnp
from jax import lax
from jax.experimental import pallas as pl
from jax.experimental.pallas import tpu as pltpu
from jax.experimental.pallas import tpu_sc as plsc

N = 10000
E = 320000
D = 128
NC = 2
NS = 16
NW = NC * NS
EPW = E // NW
K = 80
NCHUNK = EPW // K
RPT = 624
RREM = N - NS * RPT
CW = 128

_mesh = plsc.VectorSubcoreMesh(core_axis_name="c", subcore_axis_name="s")


def _lane_perm(v, idx):
    return lax.gather(
        v, idx[:, None],
        lax.GatherDimensionNumbers(offset_dims=(), collapsed_slice_dims=(0,),
                                   start_index_map=(0,)),
        slice_sizes=(1,),
        mode=lax.GatherScatterMode.PROMISE_IN_BOUNDS)


def _seg_body(h_hbm, src_hbm, dst_hbm, z_hbm, parts_out,
              src_v, dst_v, rows, sem0, acc_sh):
    cid = lax.axis_index("c")
    sid = lax.axis_index("s")
    wid = sid * NC + cid
    r0 = sid * RPT

    def _rowcopy(src_mem, dst_mem):
        pltpu.sync_copy(src_mem.at[pl.ds(r0, RPT)], dst_mem.at[pl.ds(r0, RPT)])

        @pl.when(sid == NS - 1)
        def _():
            pltpu.sync_copy(src_mem.at[pl.ds(NS * RPT, RREM)],
                            dst_mem.at[pl.ds(NS * RPT, RREM)])

    _rowcopy(z_hbm, acc_sh)
    pltpu.sync_copy(src_hbm.at[wid], src_v)
    pltpu.sync_copy(dst_hbm.at[wid], dst_v)
    plsc.subcore_barrier()

    @pl.loop(0, NCHUNK)
    def _chunk(i):
        pltpu.async_copy(h_hbm.at[src_v.at[i]], rows, sem0).wait()
        pltpu.sync_copy(rows, acc_sh.at[dst_v.at[i]], add=True)

    plsc.subcore_barrier()
    _rowcopy(acc_sh, parts_out.at[cid])


_seg_sum = functools.partial(
    pl.kernel,
    _seg_body,
    out_type=jax.ShapeDtypeStruct((NC, N, D), jnp.float32),
    mesh=_mesh,
    scratch_types=[
        pltpu.VMEM((NCHUNK, K), jnp.int32),
        pltpu.VMEM((NCHUNK, K), jnp.int32),
        pltpu.VMEM((K, D), jnp.float32),
        pltpu.SemaphoreType.DMA,
        pltpu.VMEM_SHARED((N, D), jnp.float32),
    ],
)()


def _cnt_body(dst_hbm, zc_hbm, ones_hbm, cnt_out,
              dst_v, ones_v, cnt_sh):
    cid = lax.axis_index("c")
    sid = lax.axis_index("s")
    wid = sid * NC + cid
    r0 = sid * RPT

    def _rowcopy(src_mem, dst_mem):
        pltpu.sync_copy(src_mem.at[pl.ds(r0, RPT)], dst_mem.at[pl.ds(r0, RPT)])

        @pl.when(sid == NS - 1)
        def _():
            pltpu.sync_copy(src_mem.at[pl.ds(NS * RPT, RREM)],
                            dst_mem.at[pl.ds(NS * RPT, RREM)])

    _rowcopy(zc_hbm, cnt_sh)
    pltpu.sync_copy(dst_hbm.at[wid], dst_v)
    pltpu.sync_copy(ones_hbm, ones_v)
    plsc.subcore_barrier()

    @pl.loop(0, NCHUNK)
    def _chunk(i):
        pltpu.sync_copy(ones_v, cnt_sh.at[dst_v.at[i]], add=True)

    plsc.subcore_barrier()
    _rowcopy(cnt_sh, cnt_out.at[cid])


_cnt_sum = functools.partial(
    pl.kernel,
    _cnt_body,
    out_type=jax.ShapeDtypeStruct((NC, N, CW), jnp.float32),
    mesh=_mesh,
    scratch_types=[
        pltpu.VMEM((NCHUNK, K), jnp.int32),
        pltpu.VMEM((K, CW), jnp.float32),
        pltpu.VMEM_SHARED((N, CW), jnp.float32),
    ],
)()


BN = 400


def _tc_layer_body(p_ref, c_ref, h_ref, wl_ref, bl_ref, wr_ref, o_ref):
    cnt = c_ref[0, :, 0:1] + c_ref[1, :, 0:1]
    aggr = (p_ref[0] + p_ref[1]) / jnp.maximum(cnt, 1.0)
    o_ref[...] = jnp.maximum(
        jnp.dot(aggr, wl_ref[...], preferred_element_type=jnp.float32)
        + bl_ref[...]
        + jnp.dot(h_ref[...], wr_ref[...], preferred_element_type=jnp.float32),
        0.0)


def _tc_layer(parts, cnt_parts, h, WlT, bl, WrT):
    return pl.pallas_call(
        _tc_layer_body,
        grid=(N // BN,),
        in_specs=[
            pl.BlockSpec((NC, BN, D), lambda i: (0, i, 0)),
            pl.BlockSpec((NC, BN, CW), lambda i: (0, i, 0)),
            pl.BlockSpec((BN, D), lambda i: (i, 0)),
            pl.BlockSpec((D, D), lambda i: (0, 0)),
            pl.BlockSpec((1, D), lambda i: (0, 0)),
            pl.BlockSpec((D, D), lambda i: (0, 0)),
        ],
        out_specs=pl.BlockSpec((BN, D), lambda i: (i, 0)),
        out_shape=jax.ShapeDtypeStruct((N, D), jnp.float32),
    )(parts, cnt_parts, h, WlT, bl, WrT)


def _tc_final_body(p_ref, c_ref, h_ref, wl_ref, bl_ref, wr_ref,
                   w1a_ref, w1b_ref, b1_ref, a_ref, b_ref):
    cnt = c_ref[0, :, 0:1] + c_ref[1, :, 0:1]
    aggr = (p_ref[0] + p_ref[1]) / jnp.maximum(cnt, 1.0)
    h3 = jnp.maximum(
        jnp.dot(aggr, wl_ref[...], preferred_element_type=jnp.float32)
        + bl_ref[...]
        + jnp.dot(h_ref[...], wr_ref[...], preferred_element_type=jnp.float32),
        0.0)
    a_ref[...] = jnp.dot(h3, w1a_ref[...], preferred_element_type=jnp.float32) + b1_ref[...]
    b_ref[...] = jnp.dot(h3, w1b_ref[...], preferred_element_type=jnp.float32)


def _tc_final(parts, cnt_parts, h, WlT, bl, WrT, W1aT, W1bT, b1):
    return pl.pallas_call(
        _tc_final_body,
        grid=(N // BN,),
        in_specs=[
            pl.BlockSpec((NC, BN, D), lambda i: (0, i, 0)),
            pl.BlockSpec((NC, BN, CW), lambda i: (0, i, 0)),
            pl.BlockSpec((BN, D), lambda i: (i, 0)),
            pl.BlockSpec((D, D), lambda i: (0, 0)),
            pl.BlockSpec((1, D), lambda i: (0, 0)),
            pl.BlockSpec((D, D), lambda i: (0, 0)),
            pl.BlockSpec((D, D), lambda i: (0, 0)),
            pl.BlockSpec((D, D), lambda i: (0, 0)),
            pl.BlockSpec((1, D), lambda i: (0, 0)),
        ],
        out_specs=[pl.BlockSpec((BN, D), lambda i: (i, 0)),
                   pl.BlockSpec((BN, D), lambda i: (i, 0))],
        out_shape=[jax.ShapeDtypeStruct((N, D), jnp.float32),
                   jax.ShapeDtypeStruct((N, D), jnp.float32)],
    )(parts, cnt_parts, h, WlT, bl, WrT, W1aT, W1bT, b1)


def _edge_body(a_hbm, b_hbm, src_hbm, dst_hbm, w2_hbm, b2_hbm, out_hbm,
               src_v, dst_v, a0, b0, a1, b1, w2_v, b2_v, out_v,
               sa0, sb0, sa1, sb1):
    cid = lax.axis_index("c")
    sid = lax.axis_index("s")
    wid = sid * NC + cid
    pltpu.sync_copy(src_hbm.at[wid], src_v)
    pltpu.sync_copy(dst_hbm.at[wid], dst_v)
    pltpu.sync_copy(w2_hbm, w2_v)
    pltpu.sync_copy(b2_hbm, b2_v)
    w2r = [w2_v[pl.ds(k * 16, 16)] for k in range(D // 16)]
    lane = lax.iota(jnp.int32, 16)
    pltpu.async_copy(a_hbm.at[src_v.at[0]], a0, sa0)
    pltpu.async_copy(b_hbm.at[dst_v.at[0]], b0, sb0)

    @pl.loop(0, NCHUNK)
    def _chunk(c):
        def _step(ar, br, sa, sb, oa, ob, osa, osb):
            pltpu.make_async_copy(a_hbm.at[src_v.at[c]], ar, sa).wait()
            pltpu.make_async_copy(b_hbm.at[dst_v.at[c]], br, sb).wait()

            @pl.when(c + 1 < NCHUNK)
            def _():
                pltpu.async_copy(a_hbm.at[src_v.at[c + 1]], oa, osa)
                pltpu.async_copy(b_hbm.at[dst_v.at[c + 1]], ob, osb)

            for g in range(K // 16):
                def _edge(j, acc):
                    e = g * 16 + j
                    t = jnp.zeros((16,), jnp.float32)
                    for k in range(D // 16):
                        av = ar[e, pl.ds(k * 16, 16)]
                        bv = br[e, pl.ds(k * 16, 16)]
                        t = t + jnp.maximum(av + bv, 0.0) * w2r[k]
                    for sh in (8, 4, 2, 1):
                        t = t + _lane_perm(t, jnp.bitwise_xor(lane, sh))
                    return jnp.where(lane == j, t, acc)

                acc = lax.fori_loop(0, 16, _edge, jnp.zeros((16,), jnp.float32))
                out_v[pl.ds(c * K + g * 16, 16)] = acc + b2_v[...]

        @pl.when(c % 2 == 0)
        def _even():
            _step(a0, b0, sa0, sb0, a1, b1, sa1, sb1)

        @pl.when(c % 2 == 1)
        def _odd():
            _step(a1, b1, sa1, sb1, a0, b0, sa0, sb0)

    pltpu.sync_copy(out_v, out_hbm.at[pl.ds(wid * EPW, EPW)])


_edge_mlp = functools.partial(
    pl.kernel,
    _edge_body,
    out_type=jax.ShapeDtypeStruct((E,), jnp.float32),
    mesh=_mesh,
    scratch_types=[
        pltpu.VMEM((NCHUNK, K), jnp.int32),
        pltpu.VMEM((NCHUNK, K), jnp.int32),
        pltpu.VMEM((K, D), jnp.float32),
        pltpu.VMEM((K, D), jnp.float32),
        pltpu.VMEM((K, D), jnp.float32),
        pltpu.VMEM((K, D), jnp.float32),
        pltpu.VMEM((D,), jnp.float32),
        pltpu.VMEM((16,), jnp.float32),
        pltpu.VMEM((EPW,), jnp.float32),
        pltpu.SemaphoreType.DMA,
        pltpu.SemaphoreType.DMA,
        pltpu.SemaphoreType.DMA,
        pltpu.SemaphoreType.DMA,
    ],
)()


def kernel(x, edge_index, Wl0, bl0, Wr0, Wl1, bl1, Wr1, Wl2, bl2, Wr2, W1, b1, W2, b2):
    src = edge_index[0].reshape(NW, NCHUNK, K)
    dst = edge_index[1].reshape(NW, NCHUNK, K)
    z = jnp.zeros((N, D), jnp.float32)
    zc = jnp.zeros((N, CW), jnp.float32)
    ones = jnp.ones((K, CW), jnp.float32)

    cnt_parts = _cnt_sum(dst, zc, ones)
    parts0 = _seg_sum(x, src, dst, z)
    h1 = _tc_layer(parts0, cnt_parts, x, Wl0.T, bl0.reshape(1, D), Wr0.T)
    parts1 = _seg_sum(h1, src, dst, z)
    h2 = _tc_layer(parts1, cnt_parts, h1, Wl1.T, bl1.reshape(1, D), Wr1.T)
    parts2 = _seg_sum(h2, src, dst, z)
    W1T = W1.T
    A, B = _tc_final(parts2, cnt_parts, h2, Wl2.T, bl2.reshape(1, D), Wr2.T,
                     W1T[:D], W1T[D:], b1.reshape(1, D))
    out = _edge_mlp(A, B, src, dst, W2.reshape(D), jnp.broadcast_to(b2, (16,)))
    return out
